# Initial kernel scaffold; baseline (speedup 1.0000x reference)
import jax
import jax.numpy as jnp
from jax.experimental import pallas as pl


def kernel(detections, fpn0, fpn1, fpn2, fpn3, fpn4):
    raise NotImplementedError("write your pallas kernel here")


# speedup vs baseline: 59.2852x; 59.2852x over previous
import functools

import jax
import jax.numpy as jnp
from jax import lax
from jax.experimental import pallas as pl
from jax.experimental.pallas import tpu as pltpu, tpu_sc as plsc

TOPK = 1000
CROP = 14
C = 256
NC, NS = 2, 16
NW = NC * NS
BPW = 32
NB = NW * BPW
NCHUNK = 13
PPB = NCHUNK * 16

def _crop_body(x1h, y1h, x2h, y2h, lvh, posh, atlas, out_hbm,
               x1v, y1v, x2v, y2v, lvv, posv, idx_v, w_v, rows_v, oc_v,
               zc_v, sem):
    wid = lax.axis_index("s") * NC + lax.axis_index("c")
    b0 = wid * BPW

    def zj(j, _):
        def zch(ch, _):
            zc_v[j, pl.ds(ch * 16, 16)] = jnp.zeros((16,), jnp.float32)
            return 0

        lax.fori_loop(0, 16, zch, 0)
        return 0

    lax.fori_loop(0, 16, zj, 0)
    pltpu.sync_copy(x1h.at[pl.ds(b0, BPW + 16)], x1v)
    pltpu.sync_copy(y1h.at[pl.ds(b0, BPW + 16)], y1v)
    pltpu.sync_copy(x2h.at[pl.ds(b0, BPW + 16)], x2v)
    pltpu.sync_copy(y2h.at[pl.ds(b0, BPW + 16)], y2v)
    pltpu.sync_copy(lvh.at[pl.ds(b0, BPW + 16)], lvv)
    pltpu.sync_copy(posh.at[pl.ds(b0, BPW + 16)], posv)

    def box_body(b, carry):
        pos0 = posv[pl.ds(b, 16)][0]
        lvl0 = lvv[pl.ds(b, 16)][0]
        wi0 = lax.shift_right_logical(jnp.int32(128), lvl0)
        base0 = (jnp.where(lvl0 >= 1, 16384, 0)
                 + jnp.where(lvl0 >= 2, 4096, 0)
                 + jnp.where(lvl0 >= 3, 1024, 0)
                 + jnp.where(lvl0 >= 4, 256, 0))
        hm10 = (wi0 - 1).astype(jnp.float32)
        by1 = y1v[pl.ds(b, 16)][0]
        by2 = y2v[pl.ds(b, 16)][0]
        bx1 = x1v[pl.ds(b, 16)][0]
        bx2 = x2v[pl.ds(b, 16)][0]
        wi = jnp.full((16,), 0, jnp.int32) + wi0
        base = jnp.full((16,), 0, jnp.int32) + base0
        hm1f = jnp.full((16,), 0, jnp.float32) + hm10
        y1n = (jnp.full((16,), 0, jnp.float32) + by1) / hm1f
        y2n = (jnp.full((16,), 0, jnp.float32) + by2) / hm1f
        x1n = (jnp.full((16,), 0, jnp.float32) + bx1) / hm1f
        x2n = (jnp.full((16,), 0, jnp.float32) + bx2) / hm1f
        dy = y2n - y1n
        dx = x2n - x1n
        dyr = by2 - by1
        hm10s = hm10 + 0.01
        x_any = (bx2 >= -0.01) & (bx1 <= hm10s)
        for c in range(NCHUNK):
            pvec = lax.iota(jnp.int32, 16) + (c * 16)
            pf = pvec.astype(jnp.float32)
            iyf = (pf / 14.0).astype(jnp.int32).astype(jnp.float32)
            ixf = pf - iyf * 14.0
            hs = iyf / 13.0
            ws = ixf / 13.0
            in_y = (y1n + dy * hs) * hm1f
            in_x = (x1n + dx * ws) * hm1f
            okv = ((in_y >= 0.0) & (in_y <= hm1f)
                   & (in_x >= 0.0) & (in_x <= hm1f))
            yc = jnp.minimum(jnp.maximum(in_y, 0.0), hm1f)
            xc = jnp.minimum(jnp.maximum(in_x, 0.0), hm1f)
            y0i = yc.astype(jnp.int32)
            x0i = xc.astype(jnp.int32)
            ly = yc - y0i.astype(jnp.float32)
            lx = xc - x0i.astype(jnp.float32)
            y1i = jnp.minimum(y0i + 1, wi - 1)
            x1i = jnp.minimum(x0i + 1, wi - 1)
            r0 = base + y0i * wi
            r1 = base + y1i * wi
            idx_v[0:16] = r0 + x0i
            idx_v[16:32] = r0 + x1i
            idx_v[32:48] = r1 + x0i
            idx_v[48:64] = r1 + x1i
            w_v[0:16] = ly
            w_v[16:32] = lx
            w_v[32:48] = jnp.where(okv, jnp.float32(1.0), jnp.float32(0.0))
            iy_lo = (c * 16) // 14
            iy_hi = (c * 16 + 15) // 14
            ylo = by1 + dyr * (iy_lo / 13.0)
            yhi = by1 + dyr * (iy_hi / 13.0)
            anyok = (yhi >= -0.01) & (ylo <= hm10s) & x_any

            @pl.when(anyok)
            def _():
                pltpu.async_copy(atlas.at[idx_v], rows_v, sem).wait()
                lyv = w_v[0:16]
                lxv = w_v[16:32]
                mv = w_v[32:48]

                def chloop(ch, _):
                    s = pl.ds(ch * 16, 16)
                    for j in range(16):
                        tl = rows_v[j, s]
                        tr = rows_v[16 + j, s]
                        bl = rows_v[32 + j, s]
                        br = rows_v[48 + j, s]
                        lxj = lxv[j]
                        top = tl + (tr - tl) * lxj
                        bot = bl + (br - bl) * lxj
                        oc_v[j, s] = (top + (bot - top) * lyv[j]) * mv[j]
                    return 0

                lax.fori_loop(0, 16, chloop, 0)
                pltpu.sync_copy(
                    oc_v, out_hbm.at[pl.ds(pos0 * PPB + c * 16, 16)])

            @pl.when(jnp.logical_not(anyok))
            def _():
                pltpu.sync_copy(
                    zc_v, out_hbm.at[pl.ds(pos0 * PPB + c * 16, 16)])
        return carry

    lax.fori_loop(0, BPW, box_body, 0)


_crop = functools.partial(
    pl.kernel,
    out_type=jax.ShapeDtypeStruct((NB * PPB, C), jnp.float32),
    mesh=plsc.VectorSubcoreMesh(core_axis_name="c", subcore_axis_name="s",
                                num_cores=NC, num_subcores=NS),
    scratch_types=[
        pltpu.VMEM((BPW + 16,), jnp.float32),
        pltpu.VMEM((BPW + 16,), jnp.float32),
        pltpu.VMEM((BPW + 16,), jnp.float32),
        pltpu.VMEM((BPW + 16,), jnp.float32),
        pltpu.VMEM((BPW + 16,), jnp.int32),
        pltpu.VMEM((BPW + 16,), jnp.int32),
        pltpu.VMEM((64,), jnp.int32),
        pltpu.VMEM((48,), jnp.float32),
        pltpu.VMEM((64, C), jnp.float32),
        pltpu.VMEM((16, C), jnp.float32),
        pltpu.VMEM((16, C), jnp.float32),
        pltpu.SemaphoreType.DMA,
    ],
)(_crop_body)


NPAD = NB + 16


def _scores_body(dets_ref, out_ref):
    x = dets_ref[...]
    col = lax.broadcasted_iota(jnp.int32, x.shape, 1)
    x = jnp.where((col >= 4) & (col < 84), x, -jnp.inf)
    out_ref[...] = jnp.max(x, axis=1)


def _meta_body(idx_ref, dets_ref, x1_ref, y1_ref, x2_ref, y2_ref,
               lv_ref, pos_ref, rows_ref):
    def row(i, carry):
        rows_ref[pl.ds(i, 1), :] = dets_ref[pl.ds(idx_ref[i], 1), :]
        return carry

    lax.fori_loop(0, NPAD, row, 0)
    bx1 = rows_ref[:, 0]
    by1 = rows_ref[:, 1]
    bx2 = rows_ref[:, 2]
    by2 = rows_ref[:, 3]
    x1_ref[...] = bx1
    y1_ref[...] = by1
    x2_ref[...] = bx2
    y2_ref[...] = by2
    w = bx2 - bx1
    h = by2 - by1
    size = jnp.sqrt(w * h)
    lvl = jnp.clip(jnp.floor(1.0 + jnp.log2(size / 224.0 + 1e-7)),
                   0.0, 4.0)
    iv = lax.iota(jnp.int32, NPAD)
    lvl5 = jnp.where(iv < TOPK, lvl, 5.0)
    lv_ref[...] = jnp.where(iv < TOPK, lvl, 0.0).astype(jnp.int32)
    lvf = lvl5[0:NB]
    col = lax.broadcasted_iota(jnp.int32, (NB, 128), 1)
    onehot = (col.astype(jnp.float32) == lvf[:, None]).astype(jnp.float32)
    ri = lax.broadcasted_iota(jnp.int32, (NB, NB), 0)
    ci = lax.broadcasted_iota(jnp.int32, (NB, NB), 1)
    tstrict = (ci < ri).astype(jnp.float32)
    prefix = jnp.dot(tstrict, onehot,
                     precision=lax.Precision.HIGHEST,
                     preferred_element_type=jnp.float32)
    totals = jnp.sum(onehot, axis=0)
    rk = lax.broadcasted_iota(jnp.int32, (128, 128), 0)
    cl = lax.broadcasted_iota(jnp.int32, (128, 128), 1)
    s = (rk < cl).astype(jnp.float32)
    offs = jnp.dot(totals[None, :], s,
                   precision=lax.Precision.HIGHEST,
                   preferred_element_type=jnp.float32)[0]
    pos = jnp.sum((prefix + offs[None, :]) * onehot, axis=1)
    pos_ref[0:NB] = pos.astype(jnp.int32)
    pos_ref[NB:NPAD] = jnp.zeros((16,), jnp.int32) + (NB - 1)


def _select_meta(dets_p, idx_p):
    return pl.pallas_call(
        _meta_body,
        out_shape=[
            jax.ShapeDtypeStruct((NPAD,), jnp.float32),
            jax.ShapeDtypeStruct((NPAD,), jnp.float32),
            jax.ShapeDtypeStruct((NPAD,), jnp.float32),
            jax.ShapeDtypeStruct((NPAD,), jnp.float32),
            jax.ShapeDtypeStruct((NPAD,), jnp.int32),
            jax.ShapeDtypeStruct((NPAD,), jnp.int32),
        ],
        in_specs=[
            pl.BlockSpec(memory_space=pltpu.SMEM),
            pl.BlockSpec(memory_space=pltpu.VMEM),
        ],
        scratch_shapes=[pltpu.VMEM((NPAD, 128), jnp.float32)],
    )(idx_p, dets_p)


def kernel(detections, fpn0, fpn1, fpn2, fpn3, fpn4):
    dets = detections[0]
    dets_p = jnp.pad(dets, ((0, 0), (0, 128 - dets.shape[1])))
    scores = pl.pallas_call(
        _scores_body,
        out_shape=jax.ShapeDtypeStruct((dets_p.shape[0],), jnp.float32),
    )(dets_p)
    _, idx = lax.top_k(scores, TOPK)
    idx_p = jnp.pad(idx.astype(jnp.int32), (0, NPAD - TOPK))
    x1, y1, x2, y2, lv, pos = _select_meta(dets_p, idx_p)

    atlas = jnp.concatenate(
        [f[0].reshape(-1, C) for f in (fpn0, fpn1, fpn2, fpn3, fpn4)], axis=0)

    out = _crop(x1, y1, x2, y2, lv, pos, atlas)
    rois = out.reshape(NB, PPB, C)[:TOPK, :CROP * CROP]
    return rois.reshape(1, TOPK, CROP, CROP, C)


# speedup vs baseline: 59.6916x; 1.0069x over previous
import functools

import numpy as np
import jax
import jax.numpy as jnp
from jax import lax
from jax.experimental import pallas as pl
from jax.experimental.pallas import tpu as pltpu, tpu_sc as plsc

TOPK = 1000
CROP = 14
C = 256
NC, NS = 2, 16
NW = NC * NS
BPW = 32
NB = NW * BPW
NCHUNK = 13
PPB = NCHUNK * 16

def _crop_body(x1h, y1h, x2h, y2h, lvh, posh, atlas, out_hbm,
               x1v, y1v, x2v, y2v, lvv, posv, idx_v, w_v, rows_v, oc_v,
               zc_v, sem):
    wid = lax.axis_index("s") * NC + lax.axis_index("c")
    b0 = wid * BPW

    def zj(j, _):
        def zch(ch, _):
            zc_v[j, pl.ds(ch * 16, 16)] = jnp.zeros((16,), jnp.float32)
            return 0

        lax.fori_loop(0, 16, zch, 0)
        return 0

    lax.fori_loop(0, 16, zj, 0)
    pltpu.sync_copy(x1h.at[pl.ds(b0, BPW + 16)], x1v)
    pltpu.sync_copy(y1h.at[pl.ds(b0, BPW + 16)], y1v)
    pltpu.sync_copy(x2h.at[pl.ds(b0, BPW + 16)], x2v)
    pltpu.sync_copy(y2h.at[pl.ds(b0, BPW + 16)], y2v)
    pltpu.sync_copy(lvh.at[pl.ds(b0, BPW + 16)], lvv)
    pltpu.sync_copy(posh.at[pl.ds(b0, BPW + 16)], posv)

    def box_body(b, carry):
        pos0 = posv[pl.ds(b, 16)][0]
        lvl0 = lvv[pl.ds(b, 16)][0]
        wi0 = lax.shift_right_logical(jnp.int32(128), lvl0)
        base0 = (jnp.where(lvl0 >= 1, 16384, 0)
                 + jnp.where(lvl0 >= 2, 4096, 0)
                 + jnp.where(lvl0 >= 3, 1024, 0)
                 + jnp.where(lvl0 >= 4, 256, 0))
        hm10 = (wi0 - 1).astype(jnp.float32)
        by1 = y1v[pl.ds(b, 16)][0]
        by2 = y2v[pl.ds(b, 16)][0]
        bx1 = x1v[pl.ds(b, 16)][0]
        bx2 = x2v[pl.ds(b, 16)][0]
        wi = jnp.full((16,), 0, jnp.int32) + wi0
        base = jnp.full((16,), 0, jnp.int32) + base0
        hm1f = jnp.full((16,), 0, jnp.float32) + hm10
        y1n = (jnp.full((16,), 0, jnp.float32) + by1) / hm1f
        y2n = (jnp.full((16,), 0, jnp.float32) + by2) / hm1f
        x1n = (jnp.full((16,), 0, jnp.float32) + bx1) / hm1f
        x2n = (jnp.full((16,), 0, jnp.float32) + bx2) / hm1f
        dy = y2n - y1n
        dx = x2n - x1n
        dyr = by2 - by1
        hm10s = hm10 + 0.01
        x_any = (bx2 >= -0.01) & (bx1 <= hm10s)
        for c in range(NCHUNK):
            pvec = lax.iota(jnp.int32, 16) + (c * 16)
            pf = pvec.astype(jnp.float32)
            iyf = (pf / 14.0).astype(jnp.int32).astype(jnp.float32)
            ixf = pf - iyf * 14.0
            hs = iyf / 13.0
            ws = ixf / 13.0
            in_y = (y1n + dy * hs) * hm1f
            in_x = (x1n + dx * ws) * hm1f
            okv = ((in_y >= 0.0) & (in_y <= hm1f)
                   & (in_x >= 0.0) & (in_x <= hm1f))
            yc = jnp.minimum(jnp.maximum(in_y, 0.0), hm1f)
            xc = jnp.minimum(jnp.maximum(in_x, 0.0), hm1f)
            y0i = yc.astype(jnp.int32)
            x0i = xc.astype(jnp.int32)
            ly = yc - y0i.astype(jnp.float32)
            lx = xc - x0i.astype(jnp.float32)
            y1i = jnp.minimum(y0i + 1, wi - 1)
            x1i = jnp.minimum(x0i + 1, wi - 1)
            r0 = base + y0i * wi
            r1 = base + y1i * wi
            idx_v[0:16] = r0 + x0i
            idx_v[16:32] = r0 + x1i
            idx_v[32:48] = r1 + x0i
            idx_v[48:64] = r1 + x1i
            w_v[0:16] = ly
            w_v[16:32] = lx
            w_v[32:48] = jnp.where(okv, jnp.float32(1.0), jnp.float32(0.0))
            iy_lo = (c * 16) // 14
            iy_hi = (c * 16 + 15) // 14
            ylo = by1 + dyr * (iy_lo / 13.0)
            yhi = by1 + dyr * (iy_hi / 13.0)
            anyok = (yhi >= -0.01) & (ylo <= hm10s) & x_any

            @pl.when(anyok)
            def _():
                pltpu.async_copy(atlas.at[idx_v], rows_v, sem).wait()
                lyv = w_v[0:16]
                lxv = w_v[16:32]
                mv = w_v[32:48]

                def chloop(ch, _):
                    s = pl.ds(ch * 16, 16)
                    for j in range(16):
                        tl = rows_v[j, s]
                        tr = rows_v[16 + j, s]
                        bl = rows_v[32 + j, s]
                        br = rows_v[48 + j, s]
                        lxj = lxv[j]
                        top = tl + (tr - tl) * lxj
                        bot = bl + (br - bl) * lxj
                        oc_v[j, s] = (top + (bot - top) * lyv[j]) * mv[j]
                    return 0

                lax.fori_loop(0, 16, chloop, 0)
                pltpu.sync_copy(
                    oc_v, out_hbm.at[pl.ds(pos0 * PPB + c * 16, 16)])

            @pl.when(jnp.logical_not(anyok))
            def _():
                pltpu.sync_copy(
                    zc_v, out_hbm.at[pl.ds(pos0 * PPB + c * 16, 16)])
        return carry

    lax.fori_loop(0, BPW, box_body, 0)


_crop = functools.partial(
    pl.kernel,
    out_type=jax.ShapeDtypeStruct((NB * PPB, C), jnp.float32),
    mesh=plsc.VectorSubcoreMesh(core_axis_name="c", subcore_axis_name="s",
                                num_cores=NC, num_subcores=NS),
    scratch_types=[
        pltpu.VMEM((BPW + 16,), jnp.float32),
        pltpu.VMEM((BPW + 16,), jnp.float32),
        pltpu.VMEM((BPW + 16,), jnp.float32),
        pltpu.VMEM((BPW + 16,), jnp.float32),
        pltpu.VMEM((BPW + 16,), jnp.int32),
        pltpu.VMEM((BPW + 16,), jnp.int32),
        pltpu.VMEM((64,), jnp.int32),
        pltpu.VMEM((48,), jnp.float32),
        pltpu.VMEM((64, C), jnp.float32),
        pltpu.VMEM((16, C), jnp.float32),
        pltpu.VMEM((16, C), jnp.float32),
        pltpu.SemaphoreType.DMA,
    ],
)(_crop_body)


NPAD = NB + 16

_PERM = np.concatenate([
    (np.arange(NB, dtype=np.int32).reshape(BPW, NW).T).reshape(-1),
    np.zeros(16, np.int32),
])


def _scores_body(dets_ref, out_ref):
    x = dets_ref[...]
    col = lax.broadcasted_iota(jnp.int32, x.shape, 1)
    x = jnp.where((col >= 4) & (col < 84), x, -jnp.inf)
    out_ref[...] = jnp.max(x, axis=1)


def _meta_body(idx_ref, dets_ref, x1_ref, y1_ref, x2_ref, y2_ref,
               lv_ref, pos_ref, rows_ref):
    def row(i, carry):
        rows_ref[pl.ds(i, 1), :] = dets_ref[pl.ds(idx_ref[i], 1), :]
        return carry

    lax.fori_loop(0, NPAD, row, 0)
    bx1 = rows_ref[:, 0]
    by1 = rows_ref[:, 1]
    bx2 = rows_ref[:, 2]
    by2 = rows_ref[:, 3]
    x1_ref[...] = bx1
    y1_ref[...] = by1
    x2_ref[...] = bx2
    y2_ref[...] = by2
    w = bx2 - bx1
    h = by2 - by1
    size = jnp.sqrt(w * h)
    lvl = jnp.clip(jnp.floor(1.0 + jnp.log2(size / 224.0 + 1e-7)),
                   0.0, 4.0)
    iv = lax.iota(jnp.int32, NPAD)
    lvl5 = jnp.where(iv < TOPK, lvl, 5.0)
    lv_ref[...] = jnp.where(iv < TOPK, lvl, 0.0).astype(jnp.int32)
    lvf = lvl5[0:NB]
    col = lax.broadcasted_iota(jnp.int32, (NB, 128), 1)
    onehot = (col.astype(jnp.float32) == lvf[:, None]).astype(jnp.float32)
    ri = lax.broadcasted_iota(jnp.int32, (NB, NB), 0)
    ci = lax.broadcasted_iota(jnp.int32, (NB, NB), 1)
    tstrict = (ci < ri).astype(jnp.float32)
    prefix = jnp.dot(tstrict, onehot,
                     precision=lax.Precision.HIGHEST,
                     preferred_element_type=jnp.float32)
    totals = jnp.sum(onehot, axis=0)
    rk = lax.broadcasted_iota(jnp.int32, (128, 128), 0)
    cl = lax.broadcasted_iota(jnp.int32, (128, 128), 1)
    s = (rk < cl).astype(jnp.float32)
    offs = jnp.dot(totals[None, :], s,
                   precision=lax.Precision.HIGHEST,
                   preferred_element_type=jnp.float32)[0]
    pos = jnp.sum((prefix + offs[None, :]) * onehot, axis=1)
    pos_ref[0:NB] = pos.astype(jnp.int32)
    pos_ref[NB:NPAD] = jnp.zeros((16,), jnp.int32) + (NB - 1)


def _select_meta(dets_p, idx_p):
    return pl.pallas_call(
        _meta_body,
        out_shape=[
            jax.ShapeDtypeStruct((NPAD,), jnp.float32),
            jax.ShapeDtypeStruct((NPAD,), jnp.float32),
            jax.ShapeDtypeStruct((NPAD,), jnp.float32),
            jax.ShapeDtypeStruct((NPAD,), jnp.float32),
            jax.ShapeDtypeStruct((NPAD,), jnp.int32),
            jax.ShapeDtypeStruct((NPAD,), jnp.int32),
        ],
        in_specs=[
            pl.BlockSpec(memory_space=pltpu.SMEM),
            pl.BlockSpec(memory_space=pltpu.VMEM),
        ],
        scratch_shapes=[pltpu.VMEM((NPAD, 128), jnp.float32)],
    )(idx_p, dets_p)


def kernel(detections, fpn0, fpn1, fpn2, fpn3, fpn4):
    dets = detections[0]
    dets_p = jnp.pad(dets, ((0, 0), (0, 128 - dets.shape[1])))
    scores = pl.pallas_call(
        _scores_body,
        out_shape=jax.ShapeDtypeStruct((dets_p.shape[0],), jnp.float32),
    )(dets_p)
    _, idx = lax.top_k(scores, TOPK)
    idx_p = jnp.pad(idx.astype(jnp.int32), (0, NPAD - TOPK))
    x1, y1, x2, y2, lv, pos = _select_meta(dets_p, idx_p)
    perm = jnp.asarray(_PERM)
    x1, y1, x2, y2, lv, pos = (a[perm] for a in (x1, y1, x2, y2, lv, pos))

    atlas = jnp.concatenate(
        [f[0].reshape(-1, C) for f in (fpn0, fpn1, fpn2, fpn3, fpn4)], axis=0)

    out = _crop(x1, y1, x2, y2, lv, pos, atlas)
    rois = out.reshape(NB, PPB, C)[:TOPK, :CROP * CROP]
    return rois.reshape(1, TOPK, CROP, CROP, C)


# speedup vs baseline: 71.1027x; 1.1912x over previous
import functools

import numpy as np
import jax
import jax.numpy as jnp
from jax import lax
from jax.experimental import pallas as pl
from jax.experimental.pallas import tpu as pltpu, tpu_sc as plsc

TOPK = 1000
CROP = 14
C = 256
NC, NS = 2, 16
NW = NC * NS
BPW = 32
NB = NW * BPW
NCHUNK = 13
PPB = NCHUNK * 16

def _crop_body(x1h, y1h, x2h, y2h, lvh, posh, atlas, out_hbm,
               x1v, y1v, x2v, y2v, lvv, posv, idx0_v, idx1_v, w0_v, w1_v,
               rows0_v, rows1_v, oc0_v, oc1_v,
               zc_v, gsem0, gsem1, osem0, osem1):
    rows_s = (rows0_v, rows1_v)
    oc_s = (oc0_v, oc1_v)
    gsem_s = (gsem0, gsem1)
    osem_s = (osem0, osem1)
    wid = lax.axis_index("s") * NC + lax.axis_index("c")
    b0 = wid * BPW

    def zj(j, _):
        def zch(ch, _):
            zc_v[j, pl.ds(ch * 16, 16)] = jnp.zeros((16,), jnp.float32)
            return 0

        lax.fori_loop(0, 16, zch, 0)
        return 0

    lax.fori_loop(0, 16, zj, 0)
    pltpu.sync_copy(x1h.at[pl.ds(b0, BPW + 16)], x1v)
    pltpu.sync_copy(y1h.at[pl.ds(b0, BPW + 16)], y1v)
    pltpu.sync_copy(x2h.at[pl.ds(b0, BPW + 16)], x2v)
    pltpu.sync_copy(y2h.at[pl.ds(b0, BPW + 16)], y2v)
    pltpu.sync_copy(lvh.at[pl.ds(b0, BPW + 16)], lvv)
    pltpu.sync_copy(posh.at[pl.ds(b0, BPW + 16)], posv)

    def box_body(b, carry):
        pos0 = posv[pl.ds(b, 16)][0]
        lvl0 = lvv[pl.ds(b, 16)][0]
        wi0 = lax.shift_right_logical(jnp.int32(128), lvl0)
        base0 = (jnp.where(lvl0 >= 1, 16384, 0)
                 + jnp.where(lvl0 >= 2, 4096, 0)
                 + jnp.where(lvl0 >= 3, 1024, 0)
                 + jnp.where(lvl0 >= 4, 256, 0))
        hm10 = (wi0 - 1).astype(jnp.float32)
        by1 = y1v[pl.ds(b, 16)][0]
        by2 = y2v[pl.ds(b, 16)][0]
        bx1 = x1v[pl.ds(b, 16)][0]
        bx2 = x2v[pl.ds(b, 16)][0]
        wi = jnp.full((16,), 0, jnp.int32) + wi0
        base = jnp.full((16,), 0, jnp.int32) + base0
        hm1f = jnp.full((16,), 0, jnp.float32) + hm10
        y1n = (jnp.full((16,), 0, jnp.float32) + by1) / hm1f
        y2n = (jnp.full((16,), 0, jnp.float32) + by2) / hm1f
        x1n = (jnp.full((16,), 0, jnp.float32) + bx1) / hm1f
        x2n = (jnp.full((16,), 0, jnp.float32) + bx2) / hm1f
        dy = y2n - y1n
        dx = x2n - x1n
        dyr = by2 - by1
        hm10s = hm10 + 0.01
        x_any = (bx2 >= -0.01) & (bx1 <= hm10s)
        idx_s = (idx0_v, idx1_v)
        w_s = (w0_v, w1_v)

        def compute_chunk(c):
            sl = c % 2
            iv = idx_s[sl]
            wv = w_s[sl]
            pvec = lax.iota(jnp.int32, 16) + (c * 16)
            pf = pvec.astype(jnp.float32)
            iyf = (pf / 14.0).astype(jnp.int32).astype(jnp.float32)
            ixf = pf - iyf * 14.0
            hs = iyf / 13.0
            ws = ixf / 13.0
            in_y = (y1n + dy * hs) * hm1f
            in_x = (x1n + dx * ws) * hm1f
            okv = ((in_y >= 0.0) & (in_y <= hm1f)
                   & (in_x >= 0.0) & (in_x <= hm1f))
            yc = jnp.minimum(jnp.maximum(in_y, 0.0), hm1f)
            xc = jnp.minimum(jnp.maximum(in_x, 0.0), hm1f)
            y0i = yc.astype(jnp.int32)
            x0i = xc.astype(jnp.int32)
            ly = yc - y0i.astype(jnp.float32)
            lx = xc - x0i.astype(jnp.float32)
            y1i = jnp.minimum(y0i + 1, wi - 1)
            x1i = jnp.minimum(x0i + 1, wi - 1)
            r0 = base + y0i * wi
            r1 = base + y1i * wi
            iv[0:16] = r0 + x0i
            iv[16:32] = r0 + x1i
            iv[32:48] = r1 + x0i
            iv[48:64] = r1 + x1i
            wv[0:16] = ly
            wv[16:32] = lx
            wv[32:48] = jnp.where(okv, jnp.float32(1.0), jnp.float32(0.0))

        def ok_chunk(c):
            iy_lo = (c * 16) // 14
            iy_hi = (c * 16 + 15) // 14
            ylo = by1 + dyr * (iy_lo / 13.0)
            yhi = by1 + dyr * (iy_hi / 13.0)
            return (yhi >= -0.01) & (ylo <= hm10s) & x_any

        def issue_gather(c):
            sl = c % 2

            @pl.when(ok_chunk(c))
            def _():
                pltpu.async_copy(atlas.at[idx_s[sl]], rows_s[sl], gsem_s[sl])

        compute_chunk(0)
        issue_gather(0)
        for c in range(NCHUNK):
            sl = c % 2
            if c + 1 < NCHUNK:
                compute_chunk(c + 1)
                issue_gather(c + 1)
            dst = out_hbm.at[pl.ds(pos0 * PPB + c * 16, 16)]
            if c >= 2:
                pltpu.make_async_copy(oc_s[sl], dst, osem_s[sl]).wait()
            anyok = ok_chunk(c)

            @pl.when(anyok)
            def _(c=c, sl=sl, dst=dst):
                pltpu.make_async_copy(atlas.at[idx_s[sl]], rows_s[sl],
                                      gsem_s[sl]).wait()
                rows_v = rows_s[sl]
                oc_v = oc_s[sl]
                wv = w_s[sl]
                lyv = wv[0:16]
                lxv = wv[16:32]
                mv = wv[32:48]

                def chloop(ch, _):
                    s = pl.ds(ch * 16, 16)
                    for j in range(16):
                        tl = rows_v[j, s]
                        tr = rows_v[16 + j, s]
                        bl = rows_v[32 + j, s]
                        br = rows_v[48 + j, s]
                        lxj = lxv[j]
                        top = tl + (tr - tl) * lxj
                        bot = bl + (br - bl) * lxj
                        oc_v[j, s] = (top + (bot - top) * lyv[j]) * mv[j]
                    return 0

                lax.fori_loop(0, 16, chloop, 0)
                pltpu.async_copy(oc_v, dst, osem_s[sl])

            @pl.when(jnp.logical_not(anyok))
            def _(sl=sl, dst=dst):
                pltpu.async_copy(zc_v, dst, osem_s[sl])

        pltpu.make_async_copy(
            oc_s[1], out_hbm.at[pl.ds(pos0 * PPB + 11 * 16, 16)],
            osem_s[1]).wait()
        pltpu.make_async_copy(
            oc_s[0], out_hbm.at[pl.ds(pos0 * PPB + 12 * 16, 16)],
            osem_s[0]).wait()
        return carry

    lax.fori_loop(0, BPW, box_body, 0)


_crop = functools.partial(
    pl.kernel,
    out_type=jax.ShapeDtypeStruct((NB * PPB, C), jnp.float32),
    mesh=plsc.VectorSubcoreMesh(core_axis_name="c", subcore_axis_name="s",
                                num_cores=NC, num_subcores=NS),
    scratch_types=[
        pltpu.VMEM((BPW + 16,), jnp.float32),
        pltpu.VMEM((BPW + 16,), jnp.float32),
        pltpu.VMEM((BPW + 16,), jnp.float32),
        pltpu.VMEM((BPW + 16,), jnp.float32),
        pltpu.VMEM((BPW + 16,), jnp.int32),
        pltpu.VMEM((BPW + 16,), jnp.int32),
        pltpu.VMEM((64,), jnp.int32),
        pltpu.VMEM((64,), jnp.int32),
        pltpu.VMEM((48,), jnp.float32),
        pltpu.VMEM((48,), jnp.float32),
        pltpu.VMEM((64, C), jnp.float32),
        pltpu.VMEM((64, C), jnp.float32),
        pltpu.VMEM((16, C), jnp.float32),
        pltpu.VMEM((16, C), jnp.float32),
        pltpu.VMEM((16, C), jnp.float32),
        pltpu.SemaphoreType.DMA,
        pltpu.SemaphoreType.DMA,
        pltpu.SemaphoreType.DMA,
        pltpu.SemaphoreType.DMA,
    ],
)(_crop_body)


NPAD = NB + 16

_PERM = np.concatenate([
    (np.arange(NB, dtype=np.int32).reshape(BPW, NW).T).reshape(-1),
    np.zeros(16, np.int32),
])


def _scores_body(dets_ref, out_ref):
    x = dets_ref[...]
    col = lax.broadcasted_iota(jnp.int32, x.shape, 1)
    x = jnp.where((col >= 4) & (col < 84), x, -jnp.inf)
    out_ref[...] = jnp.max(x, axis=1)


def _meta_body(idx_ref, dets_ref, x1_ref, y1_ref, x2_ref, y2_ref,
               lv_ref, pos_ref, rows_ref):
    def row(i, carry):
        rows_ref[pl.ds(i, 1), :] = dets_ref[pl.ds(idx_ref[i], 1), :]
        return carry

    lax.fori_loop(0, NPAD, row, 0)
    bx1 = rows_ref[:, 0]
    by1 = rows_ref[:, 1]
    bx2 = rows_ref[:, 2]
    by2 = rows_ref[:, 3]
    x1_ref[...] = bx1
    y1_ref[...] = by1
    x2_ref[...] = bx2
    y2_ref[...] = by2
    w = bx2 - bx1
    h = by2 - by1
    size = jnp.sqrt(w * h)
    lvl = jnp.clip(jnp.floor(1.0 + jnp.log2(size / 224.0 + 1e-7)),
                   0.0, 4.0)
    iv = lax.iota(jnp.int32, NPAD)
    lvl5 = jnp.where(iv < TOPK, lvl, 5.0)
    lv_ref[...] = jnp.where(iv < TOPK, lvl, 0.0).astype(jnp.int32)
    lvf = lvl5[0:NB]
    col = lax.broadcasted_iota(jnp.int32, (NB, 128), 1)
    onehot = (col.astype(jnp.float32) == lvf[:, None]).astype(jnp.float32)
    ri = lax.broadcasted_iota(jnp.int32, (NB, NB), 0)
    ci = lax.broadcasted_iota(jnp.int32, (NB, NB), 1)
    tstrict = (ci < ri).astype(jnp.float32)
    prefix = jnp.dot(tstrict, onehot,
                     precision=lax.Precision.HIGHEST,
                     preferred_element_type=jnp.float32)
    totals = jnp.sum(onehot, axis=0)
    rk = lax.broadcasted_iota(jnp.int32, (128, 128), 0)
    cl = lax.broadcasted_iota(jnp.int32, (128, 128), 1)
    s = (rk < cl).astype(jnp.float32)
    offs = jnp.dot(totals[None, :], s,
                   precision=lax.Precision.HIGHEST,
                   preferred_element_type=jnp.float32)[0]
    pos = jnp.sum((prefix + offs[None, :]) * onehot, axis=1)
    pos_ref[0:NB] = pos.astype(jnp.int32)
    pos_ref[NB:NPAD] = jnp.zeros((16,), jnp.int32) + (NB - 1)


def _select_meta(dets_p, idx_p):
    return pl.pallas_call(
        _meta_body,
        out_shape=[
            jax.ShapeDtypeStruct((NPAD,), jnp.float32),
            jax.ShapeDtypeStruct((NPAD,), jnp.float32),
            jax.ShapeDtypeStruct((NPAD,), jnp.float32),
            jax.ShapeDtypeStruct((NPAD,), jnp.float32),
            jax.ShapeDtypeStruct((NPAD,), jnp.int32),
            jax.ShapeDtypeStruct((NPAD,), jnp.int32),
        ],
        in_specs=[
            pl.BlockSpec(memory_space=pltpu.SMEM),
            pl.BlockSpec(memory_space=pltpu.VMEM),
        ],
        scratch_shapes=[pltpu.VMEM((NPAD, 128), jnp.float32)],
    )(idx_p, dets_p)


def kernel(detections, fpn0, fpn1, fpn2, fpn3, fpn4):
    dets = detections[0]
    dets_p = jnp.pad(dets, ((0, 0), (0, 128 - dets.shape[1])))
    scores = pl.pallas_call(
        _scores_body,
        out_shape=jax.ShapeDtypeStruct((dets_p.shape[0],), jnp.float32),
    )(dets_p)
    _, idx = lax.top_k(scores, TOPK)
    idx_p = jnp.pad(idx.astype(jnp.int32), (0, NPAD - TOPK))
    x1, y1, x2, y2, lv, pos = _select_meta(dets_p, idx_p)
    perm = jnp.asarray(_PERM)
    x1, y1, x2, y2, lv, pos = (a[perm] for a in (x1, y1, x2, y2, lv, pos))

    atlas = jnp.concatenate(
        [f[0].reshape(-1, C) for f in (fpn0, fpn1, fpn2, fpn3, fpn4)], axis=0)

    out = _crop(x1, y1, x2, y2, lv, pos, atlas)
    rois = out.reshape(NB, PPB, C)[:TOPK, :CROP * CROP]
    return rois.reshape(1, TOPK, CROP, CROP, C)
